# baseline (speedup 1.0000x reference)
import dataclasses
import functools

import jax
import jax.numpy as jnp
from jax import lax
from jax.experimental import pallas as pl
from jax.experimental.pallas import tpu as pltpu
from jax.experimental.pallas import tpu_sc as plsc

B = 16384
D = 64
NR = 1000000
NC = 2
NS = 16
L = 16
NW = NC * NS
BPW = B // NW
CHUNK = 128
NCH = BPW // CHUNK
GPC = CHUNK // L

TBL = 512

_mesh = plsc.VectorSubcoreMesh(core_axis_name="c", subcore_axis_name="s")

_cp = pltpu.CompilerParams()
if "needs_layout_passes" in pltpu.CompilerParams.__dataclass_fields__:
    _cp = dataclasses.replace(_cp, needs_layout_passes=False)
if "use_tc_tiling_on_sc" in pltpu.CompilerParams.__dataclass_fields__:
    _cp = dataclasses.replace(_cp, use_tc_tiling_on_sc=False)


def _tr_body(x_ref, o_ref):
    o_ref[...] = x_ref[...].T


def _tc_transpose(tab_t):
    return pl.pallas_call(
        _tr_body,
        grid=(pl.cdiv(NR, TBL),),
        in_specs=[pl.BlockSpec((D, TBL), lambda i: (0, i))],
        out_specs=pl.BlockSpec((TBL, D), lambda i: (i, 0)),
        out_shape=jax.ShapeDtypeStruct((NR, D), jnp.float32),
    )(tab_t)


@functools.partial(
    pl.kernel,
    mesh=_mesh,
    compiler_params=_cp,
    out_type=jax.ShapeDtypeStruct((B,), jnp.float32),
    scratch_types=[
        pltpu.VMEM((NCH, CHUNK), jnp.int32),
        pltpu.VMEM((NCH, CHUNK), jnp.int32),
        pltpu.VMEM((CHUNK, D), jnp.float32),
        pltpu.VMEM((CHUNK, D), jnp.float32),
        pltpu.VMEM((BPW,), jnp.float32),
        pltpu.SemaphoreType.DMA,
    ],
)
def _mf_dot_kernel(uid_hbm, iid_hbm, utab_hbm, itab_hbm, out_hbm,
                   uidx, iidx, urows, irows, outv, sem):
    wid = lax.axis_index("s") * NC + lax.axis_index("c")
    pltpu.sync_copy(uid_hbm.at[wid], uidx)
    pltpu.sync_copy(iid_hbm.at[wid], iidx)

    @pl.loop(0, NCH)
    def _chunk(c):
        ucp = pltpu.async_copy(utab_hbm.at[uidx.at[c]], urows, sem)
        icp = pltpu.async_copy(itab_hbm.at[iidx.at[c]], irows, sem)
        ucp.wait()
        icp.wait()

        @pl.loop(0, GPC)
        def _group(g):
            rows = g * L + lax.iota(jnp.int32, L)
            acc = jnp.zeros((L,), jnp.float32)
            for d in range(D):
                cols = jnp.full((L,), d, jnp.int32)
                u = plsc.load_gather(urows, [rows, cols])
                v = plsc.load_gather(irows, [rows, cols])
                acc = acc + u * v
            outv[pl.ds(c * CHUNK + g * L, L)] = acc

    pltpu.sync_copy(outv, out_hbm.at[pl.ds(wid * BPW, BPW)])


def kernel(user_ids, item_ids, user_table, item_table):
    ut = _tc_transpose(user_table.T)
    uid = user_ids.reshape(NW, NCH, CHUNK)
    iid = item_ids.reshape(NW, NCH, CHUNK)
    out = _mf_dot_kernel(uid, iid, ut, item_table)
    return out.reshape(B, 1)


# speedup vs baseline: 1.5098x; 1.5098x over previous
import dataclasses
import functools

import jax
import jax.numpy as jnp
from jax import lax
from jax.experimental import pallas as pl
from jax.experimental.pallas import tpu as pltpu
from jax.experimental.pallas import tpu_sc as plsc

B = 16384
D = 64
NR = 1000000
NC = 2
NS = 16
L = 16
NW = NC * NS
BPW = B // NW
CHUNK = 128
NCH = BPW // CHUNK
GPC = CHUNK // L

TBL = 2048

_mesh = plsc.VectorSubcoreMesh(core_axis_name="c", subcore_axis_name="s")

_cp = pltpu.CompilerParams()
if "needs_layout_passes" in pltpu.CompilerParams.__dataclass_fields__:
    _cp = dataclasses.replace(_cp, needs_layout_passes=False)
if "use_tc_tiling_on_sc" in pltpu.CompilerParams.__dataclass_fields__:
    _cp = dataclasses.replace(_cp, use_tc_tiling_on_sc=False)


def _tr_body(x_ref, o_ref):
    o_ref[...] = x_ref[...].reshape(D, TBL).T


def _tc_transpose(tab3):
    return pl.pallas_call(
        _tr_body,
        grid=(pl.cdiv(NR, TBL),),
        in_specs=[pl.BlockSpec((8, 8, TBL), lambda i: (0, 0, i))],
        out_specs=pl.BlockSpec((TBL, D), lambda i: (i, 0)),
        out_shape=jax.ShapeDtypeStruct((NR, D), jnp.float32),
    )(tab3)


@functools.partial(
    pl.kernel,
    mesh=_mesh,
    compiler_params=_cp,
    out_type=jax.ShapeDtypeStruct((B,), jnp.float32),
    scratch_types=[
        pltpu.VMEM((NCH, CHUNK), jnp.int32),
        pltpu.VMEM((NCH, CHUNK), jnp.int32),
        pltpu.VMEM((CHUNK, D), jnp.float32),
        pltpu.VMEM((CHUNK, D), jnp.float32),
        pltpu.VMEM((BPW,), jnp.float32),
        pltpu.SemaphoreType.DMA,
    ],
)
def _mf_dot_kernel(uid_hbm, iid_hbm, utab_hbm, itab_hbm, out_hbm,
                   uidx, iidx, urows, irows, outv, sem):
    wid = lax.axis_index("s") * NC + lax.axis_index("c")
    pltpu.sync_copy(uid_hbm.at[wid], uidx)
    pltpu.sync_copy(iid_hbm.at[wid], iidx)

    @pl.loop(0, NCH)
    def _chunk(c):
        ucp = pltpu.async_copy(utab_hbm.at[uidx.at[c]], urows, sem)
        icp = pltpu.async_copy(itab_hbm.at[iidx.at[c]], irows, sem)
        ucp.wait()
        icp.wait()

        @pl.loop(0, GPC)
        def _group(g):
            rows = g * L + lax.iota(jnp.int32, L)
            acc = jnp.zeros((L,), jnp.float32)
            for d in range(D):
                cols = jnp.full((L,), d, jnp.int32)
                u = plsc.load_gather(urows, [rows, cols])
                v = plsc.load_gather(irows, [rows, cols])
                acc = acc + u * v
            outv[pl.ds(c * CHUNK + g * L, L)] = acc

    pltpu.sync_copy(outv, out_hbm.at[pl.ds(wid * BPW, BPW)])


def kernel(user_ids, item_ids, user_table, item_table):
    ut = _tc_transpose(user_table.T.reshape(8, 8, NR))
    uid = user_ids.reshape(NW, NCH, CHUNK)
    iid = item_ids.reshape(NW, NCH, CHUNK)
    out = _mf_dot_kernel(uid, iid, ut, item_table)
    return out.reshape(B, 1)


# speedup vs baseline: 2.4769x; 1.6405x over previous
import dataclasses
import functools

import jax
import jax.numpy as jnp
from jax import lax
from jax.experimental import pallas as pl
from jax.experimental.pallas import tpu as pltpu
from jax.experimental.pallas import tpu_sc as plsc

B = 16384
D = 64
NR = 1000000
NC = 2
NS = 16
L = 16
NW = NC * NS
BPW = B // NW
CHUNK = 128
NCH = BPW // CHUNK
GPC = CHUNK // L

TBL = 2048
NTB = (NR + 2 * TBL - 1) // (2 * TBL)
NLAST = (NR + TBL - 1) // TBL - 1
UROWS = NTB * TBL

_mesh = plsc.VectorSubcoreMesh(core_axis_name="c", subcore_axis_name="s")

_cp = pltpu.CompilerParams()
if "needs_layout_passes" in pltpu.CompilerParams.__dataclass_fields__:
    _cp = dataclasses.replace(_cp, needs_layout_passes=False)
if "use_tc_tiling_on_sc" in pltpu.CompilerParams.__dataclass_fields__:
    _cp = dataclasses.replace(_cp, use_tc_tiling_on_sc=False)


def _tr_body(a_ref, b_ref, o_ref):
    o_ref[:, 0:D] = a_ref[...].reshape(D, TBL).T
    o_ref[:, D:2 * D] = b_ref[...].reshape(D, TBL).T


def _tc_transpose(tab3):
    return pl.pallas_call(
        _tr_body,
        grid=(NTB,),
        in_specs=[pl.BlockSpec((8, 8, TBL), lambda i: (0, 0, 2 * i)),
                  pl.BlockSpec(
                      (8, 8, TBL),
                      lambda i: (0, 0, jnp.minimum(2 * i + 1, NLAST)))],
        out_specs=pl.BlockSpec((TBL, 2 * D), lambda i: (i, 0)),
        out_shape=jax.ShapeDtypeStruct((UROWS, 2 * D), jnp.float32),
    )(tab3, tab3)


@functools.partial(
    pl.kernel,
    mesh=_mesh,
    compiler_params=_cp,
    out_type=jax.ShapeDtypeStruct((B,), jnp.float32),
    scratch_types=[
        pltpu.VMEM((NCH, CHUNK), jnp.int32),
        pltpu.VMEM((NCH, CHUNK), jnp.int32),
        pltpu.VMEM((NCH, CHUNK), jnp.int32),
        pltpu.VMEM((CHUNK, 2 * D), jnp.float32),
        pltpu.VMEM((CHUNK, D), jnp.float32),
        pltpu.VMEM((BPW,), jnp.float32),
        pltpu.SemaphoreType.DMA,
    ],
)
def _mf_dot_kernel(uid_hbm, iid_hbm, utab_hbm, itab_hbm, out_hbm,
                   uidx, ublk, iidx, urows, irows, outv, sem):
    wid = lax.axis_index("s") * NC + lax.axis_index("c")
    pltpu.sync_copy(uid_hbm.at[wid], uidx)
    pltpu.sync_copy(iid_hbm.at[wid], iidx)

    @pl.loop(0, NCH * CHUNK // L)
    def _blk(i):
        c = i // (CHUNK // L)
        o = (i % (CHUNK // L)) * L
        sl = pl.ds(o, L)
        u = uidx[c, sl]
        ublk[c, sl] = jnp.bitwise_or(
            jax.lax.shift_left(jax.lax.shift_right_logical(u, 12), 11),
            jnp.bitwise_and(u, TBL - 1))

    @pl.loop(0, NCH)
    def _chunk(c):
        ucp = pltpu.async_copy(utab_hbm.at[ublk.at[c]], urows, sem)
        icp = pltpu.async_copy(itab_hbm.at[iidx.at[c]], irows, sem)
        ucp.wait()
        icp.wait()

        @pl.loop(0, GPC)
        def _group(g):
            rows = g * L + lax.iota(jnp.int32, L)
            uoff = jnp.bitwise_and(
                jax.lax.shift_right_logical(uidx[c, pl.ds(g * L, L)], 11),
                1) * D
            acc = jnp.zeros((L,), jnp.float32)
            for d in range(D):
                cols = jnp.full((L,), d, jnp.int32)
                u = plsc.load_gather(urows, [rows, uoff + d])
                v = plsc.load_gather(irows, [rows, cols])
                acc = acc + u * v
            outv[pl.ds(c * CHUNK + g * L, L)] = acc

    pltpu.sync_copy(outv, out_hbm.at[pl.ds(wid * BPW, BPW)])


def kernel(user_ids, item_ids, user_table, item_table):
    ut = _tc_transpose(user_table.T.reshape(8, 8, NR))
    uid = user_ids.reshape(NW, NCH, CHUNK)
    iid = item_ids.reshape(NW, NCH, CHUNK)
    out = _mf_dot_kernel(uid, iid, ut, item_table)
    return out.reshape(B, 1)


# speedup vs baseline: 2.5757x; 1.0399x over previous
import dataclasses
import functools

import jax
import jax.numpy as jnp
from jax import lax
from jax.experimental import pallas as pl
from jax.experimental.pallas import tpu as pltpu
from jax.experimental.pallas import tpu_sc as plsc

B = 16384
D = 64
NR = 1000000
NC = 2
NS = 16
L = 16
NW = NC * NS
BPW = B // NW
CHUNK = 128
NCH = BPW // CHUNK
GPC = CHUNK // L

TBL = 2048
NTB = (NR + 2 * TBL - 1) // (2 * TBL)
NLAST = (NR + TBL - 1) // TBL - 1
UROWS = NTB * TBL

_mesh = plsc.VectorSubcoreMesh(core_axis_name="c", subcore_axis_name="s")

_cp = pltpu.CompilerParams()
if "needs_layout_passes" in pltpu.CompilerParams.__dataclass_fields__:
    _cp = dataclasses.replace(_cp, needs_layout_passes=False)
if "use_tc_tiling_on_sc" in pltpu.CompilerParams.__dataclass_fields__:
    _cp = dataclasses.replace(_cp, use_tc_tiling_on_sc=False)


def _tr_body(x_ref, o_ref):
    x = x_ref[...].reshape(D, 2 * TBL)
    t = jnp.concatenate([x[:, :TBL], x[:, TBL:]], axis=0)
    o_ref[...] = t.T


def _tc_transpose(tab3):
    return pl.pallas_call(
        _tr_body,
        grid=(NTB,),
        in_specs=[pl.BlockSpec((8, 8, 2 * TBL), lambda i: (0, 0, i))],
        out_specs=pl.BlockSpec((TBL, 2 * D), lambda i: (i, 0)),
        out_shape=jax.ShapeDtypeStruct((UROWS, 2 * D), jnp.float32),
    )(tab3)


@functools.partial(
    pl.kernel,
    mesh=_mesh,
    compiler_params=_cp,
    out_type=jax.ShapeDtypeStruct((B,), jnp.float32),
    scratch_types=[
        pltpu.VMEM((NCH, CHUNK), jnp.int32),
        pltpu.VMEM((NCH, CHUNK), jnp.int32),
        pltpu.VMEM((NCH, CHUNK), jnp.int32),
        pltpu.VMEM((CHUNK, 2 * D), jnp.float32),
        pltpu.VMEM((CHUNK, D), jnp.float32),
        pltpu.VMEM((BPW,), jnp.float32),
        pltpu.SemaphoreType.DMA,
    ],
)
def _mf_dot_kernel(uid_hbm, iid_hbm, utab_hbm, itab_hbm, out_hbm,
                   uidx, ublk, iidx, urows, irows, outv, sem):
    wid = lax.axis_index("s") * NC + lax.axis_index("c")
    pltpu.sync_copy(uid_hbm.at[wid], uidx)
    pltpu.sync_copy(iid_hbm.at[wid], iidx)

    @pl.loop(0, NCH * CHUNK // L)
    def _blk(i):
        c = i // (CHUNK // L)
        o = (i % (CHUNK // L)) * L
        sl = pl.ds(o, L)
        u = uidx[c, sl]
        ublk[c, sl] = jnp.bitwise_or(
            jax.lax.shift_left(jax.lax.shift_right_logical(u, 12), 11),
            jnp.bitwise_and(u, TBL - 1))

    @pl.loop(0, NCH)
    def _chunk(c):
        ucp = pltpu.async_copy(utab_hbm.at[ublk.at[c]], urows, sem)
        icp = pltpu.async_copy(itab_hbm.at[iidx.at[c]], irows, sem)
        ucp.wait()
        icp.wait()

        @pl.loop(0, GPC)
        def _group(g):
            rows = g * L + lax.iota(jnp.int32, L)
            uoff = jnp.bitwise_and(
                jax.lax.shift_right_logical(uidx[c, pl.ds(g * L, L)], 11),
                1) * D
            acc = jnp.zeros((L,), jnp.float32)
            for d in range(D):
                cols = jnp.full((L,), d, jnp.int32)
                u = plsc.load_gather(urows, [rows, uoff + d])
                v = plsc.load_gather(irows, [rows, cols])
                acc = acc + u * v
            outv[pl.ds(c * CHUNK + g * L, L)] = acc

    pltpu.sync_copy(outv, out_hbm.at[pl.ds(wid * BPW, BPW)])


def kernel(user_ids, item_ids, user_table, item_table):
    ut = _tc_transpose(user_table.T.reshape(8, 8, NR))
    uid = user_ids.reshape(NW, NCH, CHUNK)
    iid = item_ids.reshape(NW, NCH, CHUNK)
    out = _mf_dot_kernel(uid, iid, ut, item_table)
    return out.reshape(B, 1)


# speedup vs baseline: 2.6961x; 1.0467x over previous
import dataclasses
import functools

import jax
import jax.numpy as jnp
from jax import lax
from jax.experimental import pallas as pl
from jax.experimental.pallas import tpu as pltpu
from jax.experimental.pallas import tpu_sc as plsc

B = 16384
D = 64
NR = 1000000
NC = 2
NS = 16
L = 16
NW = NC * NS
BPW = B // NW
CHUNK = 128
NCH = BPW // CHUNK
GPC = CHUNK // L

TBL = 4096
TBLOG = TBL.bit_length() - 1
NTB = (NR + 2 * TBL - 1) // (2 * TBL)
UROWS = NTB * TBL

_mesh = plsc.VectorSubcoreMesh(core_axis_name="c", subcore_axis_name="s")

_cp = pltpu.CompilerParams()
if "needs_layout_passes" in pltpu.CompilerParams.__dataclass_fields__:
    _cp = dataclasses.replace(_cp, needs_layout_passes=False)
if "use_tc_tiling_on_sc" in pltpu.CompilerParams.__dataclass_fields__:
    _cp = dataclasses.replace(_cp, use_tc_tiling_on_sc=False)


def _tr_body(x_ref, o_ref):
    x = x_ref[...].reshape(D, 2 * TBL)
    t = jnp.concatenate([x[:, :TBL], x[:, TBL:]], axis=0)
    o_ref[...] = t.T


def _tc_transpose(tab3):
    return pl.pallas_call(
        _tr_body,
        grid=(NTB,),
        in_specs=[pl.BlockSpec((8, 8, 2 * TBL), lambda i: (0, 0, i))],
        out_specs=pl.BlockSpec((TBL, 2 * D), lambda i: (i, 0)),
        out_shape=jax.ShapeDtypeStruct((UROWS, 2 * D), jnp.float32),
    )(tab3)


@functools.partial(
    pl.kernel,
    mesh=_mesh,
    compiler_params=_cp,
    out_type=jax.ShapeDtypeStruct((B,), jnp.float32),
    scratch_types=[
        pltpu.VMEM((NCH, CHUNK), jnp.int32),
        pltpu.VMEM((NCH, CHUNK), jnp.int32),
        pltpu.VMEM((NCH, CHUNK), jnp.int32),
        pltpu.VMEM((NCH, CHUNK), jnp.int32),
        pltpu.VMEM((CHUNK, 2 * D), jnp.float32),
        pltpu.VMEM((CHUNK, 2 * D), jnp.float32),
        pltpu.VMEM((BPW,), jnp.float32),
        pltpu.SemaphoreType.DMA,
    ],
)
def _mf_dot_kernel(uid_hbm, iid_hbm, utab_hbm, itab_hbm, out_hbm,
                   uidx, ublk, iidx, iblk, urows, irows, outv, sem):
    wid = lax.axis_index("s") * NC + lax.axis_index("c")
    pltpu.sync_copy(uid_hbm.at[wid], uidx)
    pltpu.sync_copy(iid_hbm.at[wid], iidx)

    @pl.loop(0, NCH * CHUNK // L)
    def _blk(i):
        c = i // (CHUNK // L)
        o = (i % (CHUNK // L)) * L
        sl = pl.ds(o, L)
        u = uidx[c, sl]
        ublk[c, sl] = jnp.bitwise_or(
            jax.lax.shift_left(
                jax.lax.shift_right_logical(u, TBLOG + 1), TBLOG),
            jnp.bitwise_and(u, TBL - 1))
        iblk[c, sl] = jax.lax.shift_right_logical(iidx[c, sl], 1)

    @pl.loop(0, NCH)
    def _chunk(c):
        ucp = pltpu.async_copy(utab_hbm.at[ublk.at[c]], urows, sem)
        icp = pltpu.async_copy(itab_hbm.at[iblk.at[c]], irows, sem)
        ucp.wait()
        icp.wait()

        @pl.loop(0, GPC)
        def _group(g):
            rows = g * L + lax.iota(jnp.int32, L)
            uoff = jnp.bitwise_and(
                jax.lax.shift_right_logical(uidx[c, pl.ds(g * L, L)], TBLOG),
                1) * D
            ioff = jnp.bitwise_and(iidx[c, pl.ds(g * L, L)], 1) * D
            acc = jnp.zeros((L,), jnp.float32)
            for d in range(D):
                u = plsc.load_gather(urows, [rows, uoff + d])
                v = plsc.load_gather(irows, [rows, ioff + d])
                acc = acc + u * v
            outv[pl.ds(c * CHUNK + g * L, L)] = acc

    pltpu.sync_copy(outv, out_hbm.at[pl.ds(wid * BPW, BPW)])


def kernel(user_ids, item_ids, user_table, item_table):
    ut = _tc_transpose(user_table.T.reshape(8, 8, NR))
    uid = user_ids.reshape(NW, NCH, CHUNK)
    iid = item_ids.reshape(NW, NCH, CHUNK)
    it = item_table.reshape(NR // 2, 2 * D)
    out = _mf_dot_kernel(uid, iid, ut, it)
    return out.reshape(B, 1)


# speedup vs baseline: 4.7224x; 1.7516x over previous
import dataclasses
import functools

import jax
import jax.numpy as jnp
from jax import lax
from jax.experimental import pallas as pl
from jax.experimental.pallas import tpu as pltpu
from jax.experimental.pallas import tpu_sc as plsc

B = 16384
D = 64
NR = 1000000
NC = 2
NS = 16
L = 16
NW = NC * NS
BPW = B // NW
CHUNK = 128
NCH = BPW // CHUNK
GPC = CHUNK // L

TBL = 4096
TBLOG = TBL.bit_length() - 1
NTB = (NR + 2 * TBL - 1) // (2 * TBL)
UROWS = NTB * TBL

_mesh = plsc.VectorSubcoreMesh(core_axis_name="c", subcore_axis_name="s")

_cp = pltpu.CompilerParams()
if "needs_layout_passes" in pltpu.CompilerParams.__dataclass_fields__:
    _cp = dataclasses.replace(_cp, needs_layout_passes=False)
if "use_tc_tiling_on_sc" in pltpu.CompilerParams.__dataclass_fields__:
    _cp = dataclasses.replace(_cp, use_tc_tiling_on_sc=False)


def _tr_body(x_ref, o_ref):
    x = x_ref[...].reshape(D, 2 * TBL)
    t = jnp.concatenate([x[:, :TBL], x[:, TBL:]], axis=0)
    o_ref[...] = t.T


def _tc_transpose(tab3):
    return pl.pallas_call(
        _tr_body,
        grid=(NTB,),
        in_specs=[pl.BlockSpec((8, 8, 2 * TBL), lambda i: (0, 0, i))],
        out_specs=pl.BlockSpec((TBL, 2 * D), lambda i: (i, 0)),
        out_shape=jax.ShapeDtypeStruct((UROWS, 2 * D), jnp.float32),
    )(tab3)


@functools.partial(
    pl.kernel,
    mesh=_mesh,
    compiler_params=_cp,
    out_type=jax.ShapeDtypeStruct((B,), jnp.float32),
    scratch_types=[
        pltpu.VMEM((NCH, CHUNK), jnp.int32),
        pltpu.VMEM((NCH, CHUNK), jnp.int32),
        pltpu.VMEM((NCH, CHUNK), jnp.int32),
        pltpu.VMEM((NCH, CHUNK), jnp.int32),
        pltpu.VMEM((CHUNK, 2 * D), jnp.float32),
        pltpu.VMEM((CHUNK, 2 * D), jnp.float32),
        pltpu.VMEM((BPW,), jnp.float32),
        pltpu.SemaphoreType.DMA,
    ],
)
def _mf_dot_kernel(uid_hbm, iid_hbm, utab_hbm, itab_hbm, out_hbm,
                   uidx, ublk, iidx, iblk, urows, irows, outv, sem):
    wid = lax.axis_index("s") * NC + lax.axis_index("c")
    pltpu.sync_copy(uid_hbm.at[wid], uidx)
    pltpu.sync_copy(iid_hbm.at[wid], iidx)

    @pl.loop(0, NCH * CHUNK // L)
    def _blk(i):
        c = i // (CHUNK // L)
        o = (i % (CHUNK // L)) * L
        sl = pl.ds(o, L)
        u = uidx[c, sl]
        ublk[c, sl] = jnp.bitwise_or(
            jax.lax.shift_left(
                jax.lax.shift_right_logical(u, TBLOG + 1), TBLOG),
            jnp.bitwise_and(u, TBL - 1))
        v = iidx[c, sl]
        iblk[c, sl] = jnp.bitwise_or(
            jax.lax.shift_left(
                jax.lax.shift_right_logical(v, TBLOG + 1), TBLOG),
            jnp.bitwise_and(v, TBL - 1))

    @pl.loop(0, NCH)
    def _chunk(c):
        ucp = pltpu.async_copy(utab_hbm.at[ublk.at[c]], urows, sem)
        icp = pltpu.async_copy(itab_hbm.at[iblk.at[c]], irows, sem)
        ucp.wait()
        icp.wait()

        @pl.loop(0, GPC)
        def _group(g):
            rows = g * L + lax.iota(jnp.int32, L)
            uoff = jnp.bitwise_and(
                jax.lax.shift_right_logical(uidx[c, pl.ds(g * L, L)], TBLOG),
                1) * D
            ioff = jnp.bitwise_and(
                jax.lax.shift_right_logical(iidx[c, pl.ds(g * L, L)], TBLOG),
                1) * D
            acc = jnp.zeros((L,), jnp.float32)
            for d in range(D):
                u = plsc.load_gather(urows, [rows, uoff + d])
                v = plsc.load_gather(irows, [rows, ioff + d])
                acc = acc + u * v
            outv[pl.ds(c * CHUNK + g * L, L)] = acc

    pltpu.sync_copy(outv, out_hbm.at[pl.ds(wid * BPW, BPW)])


def kernel(user_ids, item_ids, user_table, item_table):
    ut = _tc_transpose(user_table.T.reshape(8, 8, NR))
    uid = user_ids.reshape(NW, NCH, CHUNK)
    iid = item_ids.reshape(NW, NCH, CHUNK)
    it = _tc_transpose(item_table.T.reshape(8, 8, NR))
    out = _mf_dot_kernel(uid, iid, ut, it)
    return out.reshape(B, 1)


# speedup vs baseline: 5.3702x; 1.1372x over previous
import dataclasses
import functools

import jax
import jax.numpy as jnp
from jax import lax
from jax.experimental import pallas as pl
from jax.experimental.pallas import tpu as pltpu
from jax.experimental.pallas import tpu_sc as plsc

B = 16384
D = 64
NR = 1000000
NC = 2
NS = 16
L = 16
NW = NC * NS
BPW = B // NW
CHUNK = 128
NCH = BPW // CHUNK
GPC = CHUNK // L

TBL = 8192
TBLOG = TBL.bit_length() - 1
NTB = (NR + 2 * TBL - 1) // (2 * TBL)
UROWS = NTB * TBL

_mesh = plsc.VectorSubcoreMesh(core_axis_name="c", subcore_axis_name="s")

_cp = pltpu.CompilerParams()
if "needs_layout_passes" in pltpu.CompilerParams.__dataclass_fields__:
    _cp = dataclasses.replace(_cp, needs_layout_passes=False)
if "use_tc_tiling_on_sc" in pltpu.CompilerParams.__dataclass_fields__:
    _cp = dataclasses.replace(_cp, use_tc_tiling_on_sc=False)


def _tr_body(x_ref, o_ref):
    x = x_ref[...].reshape(D, 2 * TBL)
    t = jnp.concatenate([x[:, :TBL], x[:, TBL:]], axis=0)
    o_ref[...] = t.T


def _tc_transpose(tab3):
    return pl.pallas_call(
        _tr_body,
        grid=(NTB,),
        in_specs=[pl.BlockSpec((8, 8, 2 * TBL), lambda i: (0, 0, i))],
        out_specs=pl.BlockSpec((TBL, 2 * D), lambda i: (i, 0)),
        out_shape=jax.ShapeDtypeStruct((UROWS, 2 * D), jnp.float32),
    )(tab3)


@functools.partial(
    pl.kernel,
    mesh=_mesh,
    compiler_params=_cp,
    out_type=jax.ShapeDtypeStruct((B,), jnp.float32),
    scratch_types=[
        pltpu.VMEM((NCH, CHUNK), jnp.int32),
        pltpu.VMEM((NCH, CHUNK), jnp.int32),
        pltpu.VMEM((NCH, CHUNK), jnp.int32),
        pltpu.VMEM((NCH, CHUNK), jnp.int32),
        pltpu.VMEM((CHUNK, 2 * D), jnp.float32),
        pltpu.VMEM((CHUNK, 2 * D), jnp.float32),
        pltpu.VMEM((BPW,), jnp.float32),
        pltpu.SemaphoreType.DMA,
    ],
)
def _mf_dot_kernel(uid_hbm, iid_hbm, utab_hbm, itab_hbm, out_hbm,
                   uidx, ublk, iidx, iblk, urows, irows, outv, sem):
    wid = lax.axis_index("s") * NC + lax.axis_index("c")
    pltpu.sync_copy(uid_hbm.at[wid], uidx)
    pltpu.sync_copy(iid_hbm.at[wid], iidx)

    @pl.loop(0, NCH * CHUNK // L)
    def _blk(i):
        c = i // (CHUNK // L)
        o = (i % (CHUNK // L)) * L
        sl = pl.ds(o, L)
        u = uidx[c, sl]
        ublk[c, sl] = jnp.bitwise_or(
            jax.lax.shift_left(
                jax.lax.shift_right_logical(u, TBLOG + 1), TBLOG),
            jnp.bitwise_and(u, TBL - 1))
        v = iidx[c, sl]
        iblk[c, sl] = jnp.bitwise_or(
            jax.lax.shift_left(
                jax.lax.shift_right_logical(v, TBLOG + 1), TBLOG),
            jnp.bitwise_and(v, TBL - 1))

    @pl.loop(0, NCH)
    def _chunk(c):
        ucp = pltpu.async_copy(utab_hbm.at[ublk.at[c]], urows, sem)
        icp = pltpu.async_copy(itab_hbm.at[iblk.at[c]], irows, sem)
        ucp.wait()
        icp.wait()

        @pl.loop(0, GPC)
        def _group(g):
            rows = g * L + lax.iota(jnp.int32, L)
            uoff = jnp.bitwise_and(
                jax.lax.shift_right_logical(uidx[c, pl.ds(g * L, L)], TBLOG),
                1) * D
            ioff = jnp.bitwise_and(
                jax.lax.shift_right_logical(iidx[c, pl.ds(g * L, L)], TBLOG),
                1) * D
            acc = jnp.zeros((L,), jnp.float32)
            for d in range(D):
                u = plsc.load_gather(urows, [rows, uoff + d])
                v = plsc.load_gather(irows, [rows, ioff + d])
                acc = acc + u * v
            outv[pl.ds(c * CHUNK + g * L, L)] = acc

    pltpu.sync_copy(outv, out_hbm.at[pl.ds(wid * BPW, BPW)])


def kernel(user_ids, item_ids, user_table, item_table):
    ut = _tc_transpose(user_table.T.reshape(8, 8, NR))
    uid = user_ids.reshape(NW, NCH, CHUNK)
    iid = item_ids.reshape(NW, NCH, CHUNK)
    it = _tc_transpose(item_table.T.reshape(8, 8, NR))
    out = _mf_dot_kernel(uid, iid, ut, it)
    return out.reshape(B, 1)


# speedup vs baseline: 5.5250x; 1.0288x over previous
import dataclasses
import functools

import jax
import jax.numpy as jnp
from jax import lax
from jax.experimental import pallas as pl
from jax.experimental.pallas import tpu as pltpu
from jax.experimental.pallas import tpu_sc as plsc

B = 16384
D = 64
NR = 1000000
NC = 2
NS = 16
L = 16
NW = NC * NS
BPW = B // NW
CHUNK = 128
NCH = BPW // CHUNK
GPC = CHUNK // L

TBL = 16384
TBLOG = TBL.bit_length() - 1
NTB = (NR + 2 * TBL - 1) // (2 * TBL)
UROWS = NTB * TBL

_mesh = plsc.VectorSubcoreMesh(core_axis_name="c", subcore_axis_name="s")

_cp = pltpu.CompilerParams()
if "needs_layout_passes" in pltpu.CompilerParams.__dataclass_fields__:
    _cp = dataclasses.replace(_cp, needs_layout_passes=False)
if "use_tc_tiling_on_sc" in pltpu.CompilerParams.__dataclass_fields__:
    _cp = dataclasses.replace(_cp, use_tc_tiling_on_sc=False)


def _tr_body(x_ref, o_ref):
    x = x_ref[...].reshape(D, 2 * TBL)
    t = jnp.concatenate([x[:, :TBL], x[:, TBL:]], axis=0)
    o_ref[...] = t.T


def _tc_transpose(tab3):
    return pl.pallas_call(
        _tr_body,
        grid=(NTB,),
        in_specs=[pl.BlockSpec((8, 8, 2 * TBL), lambda i: (0, 0, i))],
        out_specs=pl.BlockSpec((TBL, 2 * D), lambda i: (i, 0)),
        out_shape=jax.ShapeDtypeStruct((UROWS, 2 * D), jnp.float32),
    )(tab3)


@functools.partial(
    pl.kernel,
    mesh=_mesh,
    compiler_params=_cp,
    out_type=jax.ShapeDtypeStruct((B,), jnp.float32),
    scratch_types=[
        pltpu.VMEM((NCH, CHUNK), jnp.int32),
        pltpu.VMEM((NCH, CHUNK), jnp.int32),
        pltpu.VMEM((NCH, CHUNK), jnp.int32),
        pltpu.VMEM((NCH, CHUNK), jnp.int32),
        pltpu.VMEM((CHUNK, 2 * D), jnp.float32),
        pltpu.VMEM((CHUNK, 2 * D), jnp.float32),
        pltpu.VMEM((BPW,), jnp.float32),
        pltpu.SemaphoreType.DMA,
    ],
)
def _mf_dot_kernel(uid_hbm, iid_hbm, utab_hbm, itab_hbm, out_hbm,
                   uidx, ublk, iidx, iblk, urows, irows, outv, sem):
    wid = lax.axis_index("s") * NC + lax.axis_index("c")
    pltpu.sync_copy(uid_hbm.at[wid], uidx)
    pltpu.sync_copy(iid_hbm.at[wid], iidx)

    @pl.loop(0, NCH * CHUNK // L)
    def _blk(i):
        c = i // (CHUNK // L)
        o = (i % (CHUNK // L)) * L
        sl = pl.ds(o, L)
        u = uidx[c, sl]
        ublk[c, sl] = jnp.bitwise_or(
            jax.lax.shift_left(
                jax.lax.shift_right_logical(u, TBLOG + 1), TBLOG),
            jnp.bitwise_and(u, TBL - 1))
        v = iidx[c, sl]
        iblk[c, sl] = jnp.bitwise_or(
            jax.lax.shift_left(
                jax.lax.shift_right_logical(v, TBLOG + 1), TBLOG),
            jnp.bitwise_and(v, TBL - 1))

    @pl.loop(0, NCH)
    def _chunk(c):
        ucp = pltpu.async_copy(utab_hbm.at[ublk.at[c]], urows, sem)
        icp = pltpu.async_copy(itab_hbm.at[iblk.at[c]], irows, sem)
        ucp.wait()
        icp.wait()

        @pl.loop(0, GPC)
        def _group(g):
            rows = g * L + lax.iota(jnp.int32, L)
            uoff = jnp.bitwise_and(
                jax.lax.shift_right_logical(uidx[c, pl.ds(g * L, L)], TBLOG),
                1) * D
            ioff = jnp.bitwise_and(
                jax.lax.shift_right_logical(iidx[c, pl.ds(g * L, L)], TBLOG),
                1) * D
            acc = jnp.zeros((L,), jnp.float32)
            for d in range(D):
                u = plsc.load_gather(urows, [rows, uoff + d])
                v = plsc.load_gather(irows, [rows, ioff + d])
                acc = acc + u * v
            outv[pl.ds(c * CHUNK + g * L, L)] = acc

    pltpu.sync_copy(outv, out_hbm.at[pl.ds(wid * BPW, BPW)])


def kernel(user_ids, item_ids, user_table, item_table):
    ut = _tc_transpose(user_table.T.reshape(8, 8, NR))
    uid = user_ids.reshape(NW, NCH, CHUNK)
    iid = item_ids.reshape(NW, NCH, CHUNK)
    it = _tc_transpose(item_table.T.reshape(8, 8, NR))
    out = _mf_dot_kernel(uid, iid, ut, it)
    return out.reshape(B, 1)
